# baseline (speedup 1.0000x reference)
import functools
import math

import jax
import jax.numpy as jnp
from jax import lax
from jax.experimental import pallas as pl
from jax.experimental.pallas import tpu as pltpu
from jax.experimental.pallas import tpu_sc as plsc

NC = 2
NS = 16
NW = NC * NS

CHUNK = 8
NBUF = 6
LANES = 16


def _sc_fused(table, idx, pe, scale, batch):
    B = idx.shape[0]
    V, D = table.shape
    b_per_w = B // NW
    n_chunks = b_per_w // CHUNK
    pe_rows = CHUNK // batch
    mesh = plsc.VectorSubcoreMesh(core_axis_name="c", subcore_axis_name="s")

    @functools.partial(
        pl.kernel,
        mesh=mesh,
        out_type=jax.ShapeDtypeStruct((B // batch, batch, D), jnp.float32),
        scratch_types=[pltpu.VMEM((b_per_w,), jnp.int32)]
        + [pltpu.VMEM((CHUNK, D), jnp.float32)] * NBUF
        + [pltpu.VMEM((pe_rows, batch, D), jnp.float32)] * NBUF
        + [pltpu.VMEM((pe_rows, 1, D), jnp.float32)] * NBUF
        + [pltpu.SemaphoreType.DMA] * (3 * NBUF),
    )
    def k(table_hbm, idx_hbm, pe_hbm, out_hbm, idx_v, *bufs_sems):
        gbufs = bufs_sems[:NBUF]
        obufs = bufs_sems[NBUF:2 * NBUF]
        pbufs = bufs_sems[2 * NBUF:3 * NBUF]
        gsems = bufs_sems[3 * NBUF:4 * NBUF]
        psems = bufs_sems[4 * NBUF:5 * NBUF]
        wsems = bufs_sems[5 * NBUF:]
        wid = lax.axis_index("s") * NC + lax.axis_index("c")
        base = wid * b_per_w
        pe_base = wid * (b_per_w // batch)
        pltpu.sync_copy(idx_hbm.at[pl.ds(base, b_per_w)], idx_v)

        def g_copy(c, j):
            return pltpu.make_async_copy(
                table_hbm.at[idx_v.at[pl.ds(c * CHUNK, CHUNK)]],
                gbufs[j], gsems[j]
            )

        def p_copy(c, j):
            off = pl.multiple_of(pe_base + c * pe_rows, pe_rows)
            return pltpu.make_async_copy(
                pe_hbm.at[pl.ds(off, pe_rows)], pbufs[j], psems[j]
            )

        def w_copy(c, j):
            off = pl.multiple_of(pe_base + c * pe_rows, pe_rows)
            return pltpu.make_async_copy(
                obufs[j], out_hbm.at[pl.ds(off, pe_rows)], wsems[j]
            )

        def compute(j):
            gbuf, obuf, pbuf = gbufs[j], obufs[j], pbufs[j]

            @pl.loop(0, D, step=LANES)
            def _(col):
                cs = pl.ds(col, LANES)
                pvecs = [pbuf.at[pr, 0, cs][...] for pr in range(pe_rows)]
                vals = [gbuf.at[r, cs][...] for r in range(CHUNK)]
                res = [vals[r] * scale + pvecs[r // batch]
                       for r in range(CHUNK)]
                for r in range(CHUNK):
                    obuf.at[r // batch, r % batch, cs][...] = res[r]

        for c in range(NBUF):
            g_copy(c, c).start()
            p_copy(c, c).start()
        for c in range(n_chunks):
            j = c % NBUF
            if c >= NBUF:
                w_copy(c - NBUF, j).wait()
            g_copy(c, j).wait()
            p_copy(c, j).wait()
            compute(j)
            w_copy(c, j).start()
            if c + NBUF < n_chunks:
                g_copy(c + NBUF, j).start()
                p_copy(c + NBUF, j).start()
        for c in range(n_chunks - NBUF, n_chunks):
            w_copy(c, c % NBUF).wait()

    return k(table, idx, pe)


def kernel(x, emb_table, pe):
    S, B = x.shape
    V, D = emb_table.shape
    idx = x.reshape(-1).astype(jnp.int32)
    return _sc_fused(emb_table, idx, pe, math.sqrt(D), B)


# speedup vs baseline: 1.0176x; 1.0176x over previous
import functools
import math

import jax
import jax.numpy as jnp
from jax import lax
from jax.experimental import pallas as pl
from jax.experimental.pallas import tpu as pltpu
from jax.experimental.pallas import tpu_sc as plsc

NC = 2
NS = 16
NW = NC * NS

CHUNK = 16
NBUF = 3
LANES = 16


def _sc_fused(table, idx, pe, scale, batch):
    B = idx.shape[0]
    V, D = table.shape
    b_per_w = B // NW
    n_chunks = b_per_w // CHUNK
    pe_rows = CHUNK // batch
    mesh = plsc.VectorSubcoreMesh(core_axis_name="c", subcore_axis_name="s")

    @functools.partial(
        pl.kernel,
        mesh=mesh,
        out_type=jax.ShapeDtypeStruct((B // batch, batch, D), jnp.float32),
        scratch_types=[pltpu.VMEM((b_per_w,), jnp.int32)]
        + [pltpu.VMEM((CHUNK, D), jnp.float32)] * NBUF
        + [pltpu.VMEM((pe_rows, batch, D), jnp.float32)] * NBUF
        + [pltpu.VMEM((pe_rows, 1, D), jnp.float32)] * NBUF
        + [pltpu.SemaphoreType.DMA] * (3 * NBUF),
    )
    def k(table_hbm, idx_hbm, pe_hbm, out_hbm, idx_v, *bufs_sems):
        gbufs = bufs_sems[:NBUF]
        obufs = bufs_sems[NBUF:2 * NBUF]
        pbufs = bufs_sems[2 * NBUF:3 * NBUF]
        gsems = bufs_sems[3 * NBUF:4 * NBUF]
        psems = bufs_sems[4 * NBUF:5 * NBUF]
        wsems = bufs_sems[5 * NBUF:]
        wid = lax.axis_index("s") * NC + lax.axis_index("c")
        base = wid * b_per_w
        pe_base = wid * (b_per_w // batch)
        pltpu.sync_copy(idx_hbm.at[pl.ds(base, b_per_w)], idx_v)

        def g_copy(c, j):
            return pltpu.make_async_copy(
                table_hbm.at[idx_v.at[pl.ds(c * CHUNK, CHUNK)]],
                gbufs[j], gsems[j]
            )

        def p_copy(c, j):
            off = pl.multiple_of(pe_base + c * pe_rows, pe_rows)
            return pltpu.make_async_copy(
                pe_hbm.at[pl.ds(off, pe_rows)], pbufs[j], psems[j]
            )

        def w_copy(c, j):
            off = pl.multiple_of(pe_base + c * pe_rows, pe_rows)
            return pltpu.make_async_copy(
                obufs[j], out_hbm.at[pl.ds(off, pe_rows)], wsems[j]
            )

        def compute(j):
            gbuf, obuf, pbuf = gbufs[j], obufs[j], pbufs[j]

            @pl.loop(0, D, step=LANES)
            def _(col):
                cs = pl.ds(col, LANES)
                pvecs = [pbuf.at[pr, 0, cs][...] for pr in range(pe_rows)]
                vals = [gbuf.at[r, cs][...] for r in range(CHUNK)]
                res = [vals[r] * scale + pvecs[r // batch]
                       for r in range(CHUNK)]
                for r in range(CHUNK):
                    obuf.at[r // batch, r % batch, cs][...] = res[r]

        for c in range(NBUF):
            g_copy(c, c).start()
            p_copy(c, c).start()
        for c in range(n_chunks):
            j = c % NBUF
            if c >= NBUF:
                w_copy(c - NBUF, j).wait()
            g_copy(c, j).wait()
            p_copy(c, j).wait()
            compute(j)
            w_copy(c, j).start()
            if c + NBUF < n_chunks:
                g_copy(c + NBUF, j).start()
                p_copy(c + NBUF, j).start()
        for c in range(n_chunks - NBUF, n_chunks):
            w_copy(c, c % NBUF).wait()

    return k(table, idx, pe)


def kernel(x, emb_table, pe):
    S, B = x.shape
    V, D = emb_table.shape
    idx = x.reshape(-1).astype(jnp.int32)
    return _sc_fused(emb_table, idx, pe, math.sqrt(D), B)
